# Initial kernel scaffold; baseline (speedup 1.0000x reference)
import jax
import jax.numpy as jnp
from jax.experimental import pallas as pl


def kernel(x, w1_avg, w2_avg, w1_max, w2_max, conv_w, conv_b):
    raise NotImplementedError("write your pallas kernel here")


# speedup vs baseline: 1.0191x; 1.0191x over previous
import functools

import jax
import jax.numpy as jnp
from jax import lax
from jax.experimental import pallas as pl
from jax.experimental.pallas import tpu as pltpu


def _channel_gates_kernel(x_ref, w1a_ref, w2a_ref, w1m_ref, w2m_ref,
                          scale_ref, cmp_ref, *, inv_s, inv_c):
    x = x_ref[0]
    p_avg = jnp.sum(x, axis=1, keepdims=True) * inv_s
    p_max = jnp.max(x, axis=1, keepdims=True)

    def branch(p, w1_ref, w2_ref):
        h = jnp.dot(w1_ref[...], p, preferred_element_type=jnp.float32)
        h = jnp.maximum(h, 0.0)
        z = jnp.dot(w2_ref[...], h, preferred_element_type=jnp.float32)
        return jax.nn.sigmoid(z)

    scale = branch(p_avg, w1a_ref, w2a_ref) + branch(p_max, w1m_ref, w2m_ref)
    scale_ref[0] = scale

    y = x * scale
    cmp_ref[0, 0:1, :] = jnp.max(y, axis=0, keepdims=True)
    cmp_ref[0, 1:2, :] = jnp.sum(y, axis=0, keepdims=True) * inv_c


def _apply_gates_kernel(scale_ref, smap_ref, x_ref, o_ref):
    gate = jax.nn.sigmoid(smap_ref[0])
    o_ref[0] = (x_ref[0] * scale_ref[0] * gate).astype(o_ref.dtype)


def kernel(x, w1_avg, w2_avg, w1_max, w2_max, conv_w, conv_b):
    B, C, D, H, W = x.shape
    S = D * H * W
    dtype = x.dtype
    fsz = 4
    hid = w1_avg.shape[0]

    x_flat = x.reshape(B, C, S)

    scale, compress = pl.pallas_call(
        functools.partial(_channel_gates_kernel, inv_s=1.0 / S, inv_c=1.0 / C),
        out_shape=(jax.ShapeDtypeStruct((B, C, 1), jnp.float32),
                   jax.ShapeDtypeStruct((B, 2, S), jnp.float32)),
        grid=(B,),
        in_specs=[
            pl.BlockSpec((1, C, S), lambda b: (b, 0, 0)),
            pl.BlockSpec((hid, C), lambda b: (0, 0)),
            pl.BlockSpec((C, hid), lambda b: (0, 0)),
            pl.BlockSpec((hid, C), lambda b: (0, 0)),
            pl.BlockSpec((C, hid), lambda b: (0, 0)),
        ],
        out_specs=(pl.BlockSpec((1, C, 1), lambda b: (b, 0, 0)),
                   pl.BlockSpec((1, 2, S), lambda b: (b, 0, 0))),
        compiler_params=pltpu.CompilerParams(
            dimension_semantics=("parallel",),
            vmem_limit_bytes=48 << 20,
        ),
        cost_estimate=pl.CostEstimate(
            flops=5 * B * C * S + 4 * B * C * hid,
            transcendentals=2 * B * C,
            bytes_accessed=(B * C * S + 2 * B * S + B * C) * fsz),
    )(x_flat, w1_avg, w2_avg, w1_max, w2_max)

    comp = compress.reshape(B, 2, D, H, W)
    s_map = lax.conv_general_dilated(
        comp, conv_w.astype(jnp.float32),
        window_strides=(1, 1, 1), padding=[(3, 3)] * 3,
        dimension_numbers=("NCDHW", "OIDHW", "NCDHW"))
    s_map = s_map + conv_b.reshape(1, 1, 1, 1, 1)
    s_map_flat = s_map.reshape(B, 1, S)

    out_flat = pl.pallas_call(
        _apply_gates_kernel,
        out_shape=jax.ShapeDtypeStruct((B, C, S), dtype),
        grid=(B,),
        in_specs=[
            pl.BlockSpec((1, C, 1), lambda b: (b, 0, 0)),
            pl.BlockSpec((1, 1, S), lambda b: (b, 0, 0)),
            pl.BlockSpec((1, C, S), lambda b: (b, 0, 0)),
        ],
        out_specs=pl.BlockSpec((1, C, S), lambda b: (b, 0, 0)),
        compiler_params=pltpu.CompilerParams(
            dimension_semantics=("parallel",),
            vmem_limit_bytes=48 << 20,
        ),
        cost_estimate=pl.CostEstimate(
            flops=2 * B * C * S, transcendentals=B * S,
            bytes_accessed=(2 * B * C * S + B * S + B * C) * fsz),
    )(scale, s_map_flat, x_flat)

    return out_flat.reshape(B, C, D, H, W)


# speedup vs baseline: 4.2629x; 4.1828x over previous
import functools

import jax
import jax.numpy as jnp
from jax.experimental import pallas as pl
from jax.experimental.pallas import tpu as pltpu


def _round_up(x, m):
    return (x + m - 1) // m * m


def _channel_gates_kernel(x_ref, w1a_ref, w2a_ref, w1m_ref, w2m_ref,
                          scale_ref, cmp_ref, *, inv_s, inv_c):
    x = x_ref[0]
    p_avg = jnp.sum(x, axis=1, keepdims=True) * inv_s
    p_max = jnp.max(x, axis=1, keepdims=True)

    def branch(p, w1_ref, w2_ref):
        h = jnp.dot(w1_ref[...], p, preferred_element_type=jnp.float32)
        h = jnp.maximum(h, 0.0)
        z = jnp.dot(w2_ref[...], h, preferred_element_type=jnp.float32)
        return jax.nn.sigmoid(z)

    scale = branch(p_avg, w1a_ref, w2a_ref) + branch(p_max, w1m_ref, w2m_ref)
    scale_ref[0] = scale

    y = x * scale
    cmp_ref[0, 0:1, :] = jnp.max(y, axis=0, keepdims=True)
    cmp_ref[0, 1:2, :] = jnp.sum(y, axis=0, keepdims=True) * inv_c


def _conv_apply_kernel(scale_ref, cpad_ref, t_ref, b_ref, x_ref, o_ref,
                       *, D, H, W, K):
    Dp, Hp = D + K - 1, _round_up(H + K - 1, 8)
    HW = H * W

    cp = cpad_ref[0].reshape(Dp * Hp, cpad_ref.shape[-1])
    s1 = jnp.dot(cp, t_ref[...], preferred_element_type=jnp.float32)
    s1 = s1.reshape(Dp, Hp, K * K * W)

    acc = jnp.full((D, H, W), b_ref[0, 0], jnp.float32)
    for kd in range(K):
        for kh in range(K):
            off = (kd * K + kh) * W
            acc += s1[kd:kd + D, kh:kh + H, off:off + W]

    gate = jax.nn.sigmoid(acc).reshape(1, D, HW)
    x = x_ref[0]
    o_ref[0] = (x * scale_ref[0][:, :, None] * gate).astype(o_ref.dtype)


def kernel(x, w1_avg, w2_avg, w1_max, w2_max, conv_w, conv_b):
    B, C, D, H, W = x.shape
    S = D * H * W
    dtype = x.dtype
    fsz = 4
    hid = w1_avg.shape[0]
    K = conv_w.shape[-1]
    P = K // 2
    Dp, Hp = D + K - 1, _round_up(H + K - 1, 8)

    x_flat = x.reshape(B, C, S)

    scale, compress = pl.pallas_call(
        functools.partial(_channel_gates_kernel, inv_s=1.0 / S, inv_c=1.0 / C),
        out_shape=(jax.ShapeDtypeStruct((B, C, 1), jnp.float32),
                   jax.ShapeDtypeStruct((B, 2, S), jnp.float32)),
        grid=(B,),
        in_specs=[
            pl.BlockSpec((1, C, S), lambda b: (b, 0, 0)),
            pl.BlockSpec((hid, C), lambda b: (0, 0)),
            pl.BlockSpec((C, hid), lambda b: (0, 0)),
            pl.BlockSpec((hid, C), lambda b: (0, 0)),
            pl.BlockSpec((C, hid), lambda b: (0, 0)),
        ],
        out_specs=(pl.BlockSpec((1, C, 1), lambda b: (b, 0, 0)),
                   pl.BlockSpec((1, 2, S), lambda b: (b, 0, 0))),
        compiler_params=pltpu.CompilerParams(
            dimension_semantics=("parallel",),
            vmem_limit_bytes=48 << 20,
        ),
        cost_estimate=pl.CostEstimate(
            flops=5 * B * C * S + 4 * B * C * hid,
            transcendentals=2 * B * C,
            bytes_accessed=(B * C * S + 2 * B * S + B * C) * fsz),
    )(x_flat, w1_avg, w2_avg, w1_max, w2_max)

    comp = compress.reshape(B, 2, D, H, W)
    cpad = jnp.pad(comp, ((0, 0), (0, 0), (P, K - 1 - P),
                          (P, Hp - H - P), (0, 0)))
    cpad = cpad.transpose(0, 2, 3, 1, 4).reshape(B, Dp, Hp, 2 * W)

    w_in = jnp.arange(W)[:, None]
    w_out = jnp.arange(W)[None, :]
    kw_idx = w_in - w_out + P
    band = jnp.take(conv_w[0].astype(jnp.float32), kw_idx.clip(0, K - 1),
                    axis=-1)
    band = band * ((kw_idx >= 0) & (kw_idx < K))[None, None, None]
    tmat = band.transpose(0, 3, 1, 2, 4).reshape(2 * W, K * K * W)

    x4 = x_flat.reshape(B, C, D, H * W)
    out4 = pl.pallas_call(
        functools.partial(_conv_apply_kernel, D=D, H=H, W=W, K=K),
        out_shape=jax.ShapeDtypeStruct((B, C, D, H * W), dtype),
        grid=(B,),
        in_specs=[
            pl.BlockSpec((1, C, 1), lambda b: (b, 0, 0)),
            pl.BlockSpec((1, Dp, Hp, 2 * W), lambda b: (b, 0, 0, 0)),
            pl.BlockSpec((2 * W, K * K * W), lambda b: (0, 0)),
            pl.BlockSpec((1, 1), lambda b: (0, 0)),
            pl.BlockSpec((1, C, D, H * W), lambda b: (b, 0, 0, 0)),
        ],
        out_specs=pl.BlockSpec((1, C, D, H * W), lambda b: (b, 0, 0, 0)),
        compiler_params=pltpu.CompilerParams(
            dimension_semantics=("parallel",),
            vmem_limit_bytes=52 << 20,
        ),
        cost_estimate=pl.CostEstimate(
            flops=2 * B * C * S + 2 * B * Dp * Hp * 2 * W * K * K * W,
            transcendentals=B * S,
            bytes_accessed=(2 * B * C * S + B * S + B * C) * fsz),
    )(scale, cpad, tmat, conv_b.reshape(1, 1).astype(jnp.float32), x4)

    return out4.reshape(B, C, D, H, W)
